# baseline (speedup 1.0000x reference)
import functools

import jax
import jax.numpy as jnp
from jax import lax
from jax.experimental import pallas as pl
from jax.experimental.pallas import tpu as pltpu
from jax.experimental.pallas import tpu_sc as plsc

_NUM_CORES = 2
_NUM_SUBCORES = 16
_NW = _NUM_CORES * _NUM_SUBCORES
_LANES = 16


def _table_body(tok_ref, pos_ref, seg_ref, gamma_ref, beta_ref, out_ref):
    V, D = tok_ref.shape
    M = pos_ref.shape[0]
    G = seg_ref.shape[0]
    e = (tok_ref[:][:, None, None, :]
         + pos_ref[:][None, :, None, :]
         + seg_ref[:][None, None, :, :])
    e = e.reshape(V * M * G, D)
    mean = jnp.mean(e, axis=-1, keepdims=True)
    c = e - mean
    var = jnp.mean(c * c, axis=-1, keepdims=True)
    out_ref[:] = c * lax.rsqrt(var + 1e-5) * gamma_ref[:] + beta_ref[:]


def _build_table(tok_embed, pos_embed, seg_embed, gamma, beta):
    V, D = tok_embed.shape
    M = pos_embed.shape[0]
    G = seg_embed.shape[0]
    return pl.pallas_call(
        _table_body,
        out_shape=jax.ShapeDtypeStruct((V * M * G, D), jnp.float32),
    )(tok_embed, pos_embed, seg_embed, gamma.reshape(1, D), beta.reshape(1, D))


def _make_sc_gather(Bt, S, D, M, G, rows_per_chunk):
    B = Bt * S
    n_chunk = rows_per_chunk * S
    b_per_w = B // _NW
    n_iters = b_per_w // n_chunk
    n_pairs = n_iters // 2
    sub = D // 128
    n_sub = n_chunk * sub
    mesh = plsc.VectorSubcoreMesh(core_axis_name="c", subcore_axis_name="s")

    @functools.partial(
        pl.kernel,
        mesh=mesh,
        out_type=jax.ShapeDtypeStruct((B * sub, 128), jnp.float32),
        compiler_params=pltpu.CompilerParams(needs_layout_passes=False),
        scratch_types=[
            pltpu.VMEM((b_per_w,), jnp.int32),
            pltpu.VMEM((b_per_w,), jnp.int32),
            pltpu.VMEM((b_per_w * sub,), jnp.int32),
            pltpu.VMEM((n_sub, 128), jnp.float32),
            pltpu.VMEM((n_sub, 128), jnp.float32),
            pltpu.SemaphoreType.DMA,
            pltpu.SemaphoreType.DMA,
            pltpu.SemaphoreType.DMA,
            pltpu.SemaphoreType.DMA,
        ],
    )
    def sc_gather(x_hbm, seg_hbm, table_hbm, out_hbm, x_v, seg_v, idx_v,
                  rows0_v, rows1_v, g0, g1, s0, s1):
        wid = lax.axis_index("s") * _NUM_CORES + lax.axis_index("c")
        base = wid * b_per_w
        pltpu.sync_copy(x_hbm.at[pl.ds(base, b_per_w)], x_v)
        pltpu.sync_copy(seg_hbm.at[pl.ds(base, b_per_w)], seg_v)

        def idx_body(i, _):
            lane = lax.broadcasted_iota(jnp.int32, (_LANES,), 0)
            u = i * _LANES + lane
            tok = lax.div(u, sub)
            lt = lax.rem(u, sub)
            p = lax.rem(base + tok, M)
            xx = plsc.load_gather(x_v, [tok])
            ss = plsc.load_gather(seg_v, [tok])
            idx_v[pl.ds(i * _LANES, _LANES)] = (
                (xx * (M * G) + p * G + ss) * sub + lt)
            return 0

        lax.fori_loop(0, b_per_w * sub // _LANES, idx_body, 0)

        def g_start(k, rbuf, sem):
            pltpu.async_copy(table_hbm.at[idx_v.at[pl.ds(k * n_sub, n_sub)]],
                             rbuf, sem)

        def g_wait(rbuf, sem):
            pltpu.make_async_copy(
                table_hbm.at[idx_v.at[pl.ds(0, n_sub)]], rbuf, sem).wait()

        def s_start(k, rbuf, sem):
            pltpu.async_copy(
                rbuf, out_hbm.at[pl.ds(base * sub + k * n_sub, n_sub)], sem)

        def s_wait(rbuf, sem):
            pltpu.make_async_copy(
                rbuf, out_hbm.at[pl.ds(base * sub, n_sub)], sem).wait()

        g_start(0, rows0_v, g0)
        g_start(1, rows1_v, g1)

        def pair_body(i, _):
            a = 2 * i
            g_wait(rows0_v, g0)
            s_start(a, rows0_v, s0)
            g_wait(rows1_v, g1)
            s_start(a + 1, rows1_v, s1)

            @pl.when(i < n_pairs - 1)
            def _():
                s_wait(rows0_v, s0)
                g_start(a + 2, rows0_v, g0)
                s_wait(rows1_v, s1)
                g_start(a + 3, rows1_v, g1)

            return 0

        lax.fori_loop(0, n_pairs, pair_body, 0)
        s_wait(rows0_v, s0)
        s_wait(rows1_v, s1)

    return sc_gather


def _assemble_body(in_ref, out_ref):
    Bb, S, D = out_ref.shape
    sub = D // 128
    v = in_ref[:].reshape(Bb * S, sub, 128)
    for lt in range(sub):
        out_ref[:, :, pl.ds(lt * 128, 128)] = (
            v[:, lt, :].reshape(Bb, S, 128))


def _assemble(y, Bt, S, D, block_rows):
    sub = D // 128
    grid = Bt // block_rows
    return pl.pallas_call(
        _assemble_body,
        grid=(grid,),
        in_specs=[pl.BlockSpec((block_rows * S * sub, 128),
                               lambda g: (g, 0))],
        out_specs=pl.BlockSpec((block_rows, S, D), lambda g: (g, 0, 0)),
        out_shape=jax.ShapeDtypeStruct((Bt, S, D), jnp.float32),
    )(y)


def kernel(x, seg, tok_embed, pos_embed, seg_embed, gamma, beta):
    Bt, S = x.shape
    V, D = tok_embed.shape
    M = pos_embed.shape[0]
    G = seg_embed.shape[0]
    B = Bt * S

    table = _build_table(tok_embed, pos_embed, seg_embed, gamma, beta)
    table6 = table.reshape(V * M * G * (D // 128), 128)
    x_flat = x.reshape(B).astype(jnp.int32)
    seg_flat = seg.reshape(B).astype(jnp.int32)
    y = _make_sc_gather(Bt, S, D, M, G, rows_per_chunk=2)(
        x_flat, seg_flat, table6)
    return _assemble(y, Bt, S, D, block_rows=8)


# speedup vs baseline: 1.0050x; 1.0050x over previous
import functools

import jax
import jax.numpy as jnp
from jax import lax
from jax.experimental import pallas as pl
from jax.experimental.pallas import tpu as pltpu
from jax.experimental.pallas import tpu_sc as plsc

_NUM_CORES = 2
_NUM_SUBCORES = 16
_NW = _NUM_CORES * _NUM_SUBCORES
_LANES = 16


def _table_body(tok_ref, pos_ref, seg_ref, gamma_ref, beta_ref, out_ref):
    V, D = tok_ref.shape
    M = pos_ref.shape[0]
    G = seg_ref.shape[0]
    e = (tok_ref[:][:, None, None, :]
         + pos_ref[:][None, :, None, :]
         + seg_ref[:][None, None, :, :])
    e = e.reshape(V * M * G, D)
    mean = jnp.mean(e, axis=-1, keepdims=True)
    c = e - mean
    var = jnp.mean(c * c, axis=-1, keepdims=True)
    out_ref[:] = c * lax.rsqrt(var + 1e-5) * gamma_ref[:] + beta_ref[:]


def _build_table(tok_embed, pos_embed, seg_embed, gamma, beta):
    V, D = tok_embed.shape
    M = pos_embed.shape[0]
    G = seg_embed.shape[0]
    return pl.pallas_call(
        _table_body,
        out_shape=jax.ShapeDtypeStruct((V * M * G, D), jnp.float32),
    )(tok_embed, pos_embed, seg_embed, gamma.reshape(1, D), beta.reshape(1, D))


def _make_sc_gather(Bt, S, D, M, G):
    Sp = (S + 7) // 8 * 8
    sub = D // 128
    spr = Sp * sub
    rows_per_w = Bt // _NW
    sub_per_w = rows_per_w * spr
    rpc = 2
    n_sub = rpc * spr
    n_iters = rows_per_w // rpc
    n_pairs = n_iters // 2
    mesh = plsc.VectorSubcoreMesh(core_axis_name="c", subcore_axis_name="s")

    @functools.partial(
        pl.kernel,
        mesh=mesh,
        out_type=jax.ShapeDtypeStruct((Bt * spr, 128), jnp.float32),
        compiler_params=pltpu.CompilerParams(needs_layout_passes=False),
        scratch_types=[
            pltpu.VMEM((rows_per_w * S,), jnp.int32),
            pltpu.VMEM((rows_per_w * S,), jnp.int32),
            pltpu.VMEM((sub_per_w,), jnp.int32),
            pltpu.VMEM((n_sub, 128), jnp.float32),
            pltpu.VMEM((n_sub, 128), jnp.float32),
            pltpu.SemaphoreType.DMA,
            pltpu.SemaphoreType.DMA,
            pltpu.SemaphoreType.DMA,
            pltpu.SemaphoreType.DMA,
        ],
    )
    def sc_gather(x_hbm, seg_hbm, table_hbm, out_hbm, x_v, seg_v, idx_v,
                  rows0_v, rows1_v, g0, g1, s0, s1):
        wid = lax.axis_index("s") * _NUM_CORES + lax.axis_index("c")
        base = wid * rows_per_w * S
        base_sub = wid * sub_per_w
        pltpu.sync_copy(x_hbm.at[pl.ds(base, rows_per_w * S)], x_v)
        pltpu.sync_copy(seg_hbm.at[pl.ds(base, rows_per_w * S)], seg_v)

        def idx_body(i, _):
            lane = lax.broadcasted_iota(jnp.int32, (_LANES,), 0)
            u = i * _LANES + lane
            r = lax.div(u, spr)
            w = lax.rem(u, spr)
            lt = lax.rem(lax.div(w, 8), sub)
            p = jnp.minimum(lax.div(w, sub * 8) * 8 + lax.rem(w, 8), S - 1)
            tok = r * S + p
            xx = plsc.load_gather(x_v, [tok])
            ss = plsc.load_gather(seg_v, [tok])
            idx_v[pl.ds(i * _LANES, _LANES)] = (
                (xx * (M * G) + p * G + ss) * sub + lt)
            return 0

        lax.fori_loop(0, sub_per_w // _LANES, idx_body, 0)

        def g_start(k, rbuf, sem):
            pltpu.async_copy(table_hbm.at[idx_v.at[pl.ds(k * n_sub, n_sub)]],
                             rbuf, sem)

        def g_wait(rbuf, sem):
            pltpu.make_async_copy(
                table_hbm.at[idx_v.at[pl.ds(0, n_sub)]], rbuf, sem).wait()

        def s_start(k, rbuf, sem):
            pltpu.async_copy(
                rbuf, out_hbm.at[pl.ds(base_sub + k * n_sub, n_sub)], sem)

        def s_wait(rbuf, sem):
            pltpu.make_async_copy(
                rbuf, out_hbm.at[pl.ds(base_sub, n_sub)], sem).wait()

        g_start(0, rows0_v, g0)
        g_start(1, rows1_v, g1)

        def pair_body(i, _):
            a = 2 * i
            g_wait(rows0_v, g0)
            s_start(a, rows0_v, s0)
            g_wait(rows1_v, g1)
            s_start(a + 1, rows1_v, s1)

            @pl.when(i < n_pairs - 1)
            def _():
                s_wait(rows0_v, s0)
                g_start(a + 2, rows0_v, g0)
                s_wait(rows1_v, s1)
                g_start(a + 3, rows1_v, g1)

            return 0

        lax.fori_loop(0, n_pairs, pair_body, 0)
        s_wait(rows0_v, s0)
        s_wait(rows1_v, s1)

    return sc_gather


def _assemble_body(in_ref, out_ref):
    Bb, S, D = out_ref.shape
    sub = D // 128
    St = (S + 7) // 8
    v = in_ref[:].reshape(Bb, St, sub, 8, 128)
    for lt in range(sub):
        out_ref[:, :, pl.ds(lt * 128, 128)] = (
            v[:, :, lt, :, :].reshape(Bb, St * 8, 128)[:, :S, :])


def _assemble(y, Bt, S, D, block_rows):
    spr = ((S + 7) // 8 * 8) * (D // 128)
    grid = Bt // block_rows
    return pl.pallas_call(
        _assemble_body,
        grid=(grid,),
        in_specs=[pl.BlockSpec((block_rows * spr, 128), lambda g: (g, 0))],
        out_specs=pl.BlockSpec((block_rows, S, D), lambda g: (g, 0, 0)),
        out_shape=jax.ShapeDtypeStruct((Bt, S, D), jnp.float32),
    )(y)


def kernel(x, seg, tok_embed, pos_embed, seg_embed, gamma, beta):
    Bt, S = x.shape
    V, D = tok_embed.shape
    M = pos_embed.shape[0]
    G = seg_embed.shape[0]
    B = Bt * S

    table = _build_table(tok_embed, pos_embed, seg_embed, gamma, beta)
    table6 = table.reshape(V * M * G * (D // 128), 128)
    x_flat = x.reshape(B).astype(jnp.int32)
    seg_flat = seg.reshape(B).astype(jnp.int32)
    y = _make_sc_gather(Bt, S, D, M, G)(x_flat, seg_flat, table6)
    return _assemble(y, Bt, S, D, block_rows=8)


# speedup vs baseline: 1.2287x; 1.2225x over previous
import functools

import jax
import jax.numpy as jnp
from jax import lax
from jax.experimental import pallas as pl
from jax.experimental.pallas import tpu as pltpu
from jax.experimental.pallas import tpu_sc as plsc

_NUM_CORES = 2
_NUM_SUBCORES = 16
_NW = _NUM_CORES * _NUM_SUBCORES
_LANES = 16


def _table_body(tok_ref, pos_ref, seg_ref, gamma_ref, beta_ref, out_ref):
    V, D = tok_ref.shape
    M = pos_ref.shape[0]
    G = seg_ref.shape[0]
    e = (tok_ref[:][:, None, None, :]
         + pos_ref[:][None, :, None, :]
         + seg_ref[:][None, None, :, :])
    e = e.reshape(V * M * G, D)
    mean = jnp.mean(e, axis=-1, keepdims=True)
    c = e - mean
    var = jnp.mean(c * c, axis=-1, keepdims=True)
    out_ref[:] = c * lax.rsqrt(var + 1e-5) * gamma_ref[:] + beta_ref[:]


def _build_table(tok_embed, pos_embed, seg_embed, gamma, beta):
    V, D = tok_embed.shape
    M = pos_embed.shape[0]
    G = seg_embed.shape[0]
    return pl.pallas_call(
        _table_body,
        out_shape=jax.ShapeDtypeStruct((V * M * G, D), jnp.float32),
    )(tok_embed, pos_embed, seg_embed, gamma.reshape(1, D), beta.reshape(1, D))


def _make_sc_gather(B, D, M, G, n_chunk):
    b_per_w = B // _NW
    n_iters = b_per_w // n_chunk
    n_pairs = n_iters // 2
    mesh = plsc.VectorSubcoreMesh(core_axis_name="c", subcore_axis_name="s")

    @functools.partial(
        pl.kernel,
        mesh=mesh,
        out_type=jax.ShapeDtypeStruct((B, D), jnp.float32),
        scratch_types=[
            pltpu.VMEM((b_per_w,), jnp.int32),
            pltpu.VMEM((b_per_w,), jnp.int32),
            pltpu.VMEM((b_per_w,), jnp.int32),
            pltpu.VMEM((n_chunk, D), jnp.float32),
            pltpu.VMEM((n_chunk, D), jnp.float32),
            pltpu.SemaphoreType.DMA,
            pltpu.SemaphoreType.DMA,
            pltpu.SemaphoreType.DMA,
            pltpu.SemaphoreType.DMA,
        ],
    )
    def sc_gather(x_hbm, seg_hbm, table_hbm, out_hbm, x_v, seg_v, idx_v,
                  rows0_v, rows1_v, g0, g1, s0, s1):
        wid = lax.axis_index("s") * _NUM_CORES + lax.axis_index("c")
        base = wid * b_per_w
        pltpu.sync_copy(x_hbm.at[pl.ds(base, b_per_w)], x_v)
        pltpu.sync_copy(seg_hbm.at[pl.ds(base, b_per_w)], seg_v)

        def idx_body(i, _):
            lane = lax.broadcasted_iota(jnp.int32, (_LANES,), 0)
            j = base + i * _LANES + lane
            p = lax.rem(j, M)
            xx = x_v[pl.ds(i * _LANES, _LANES)]
            ss = seg_v[pl.ds(i * _LANES, _LANES)]
            idx_v[pl.ds(i * _LANES, _LANES)] = xx * (M * G) + p * G + ss
            return 0

        lax.fori_loop(0, b_per_w // _LANES, idx_body, 0)

        def g_start(k, rbuf, sem):
            pltpu.async_copy(table_hbm.at[idx_v.at[pl.ds(k * n_chunk, n_chunk)]],
                             rbuf, sem)

        def g_wait(rbuf, sem):
            pltpu.make_async_copy(
                table_hbm.at[idx_v.at[pl.ds(0, n_chunk)]], rbuf, sem).wait()

        def s_start(k, rbuf, sem):
            pltpu.async_copy(rbuf, out_hbm.at[pl.ds(base + k * n_chunk, n_chunk)],
                             sem)

        def s_wait(rbuf, sem):
            pltpu.make_async_copy(
                rbuf, out_hbm.at[pl.ds(base, n_chunk)], sem).wait()

        g_start(0, rows0_v, g0)
        g_start(1, rows1_v, g1)

        def pair_body(i, _):
            a = 2 * i
            g_wait(rows0_v, g0)
            s_start(a, rows0_v, s0)
            g_wait(rows1_v, g1)
            s_start(a + 1, rows1_v, s1)

            @pl.when(i < n_pairs - 1)
            def _():
                s_wait(rows0_v, s0)
                g_start(a + 2, rows0_v, g0)
                s_wait(rows1_v, s1)
                g_start(a + 3, rows1_v, g1)

            return 0

        lax.fori_loop(0, n_pairs, pair_body, 0)
        s_wait(rows0_v, s0)
        s_wait(rows1_v, s1)

    return sc_gather


def kernel(x, seg, tok_embed, pos_embed, seg_embed, gamma, beta):
    Bt, S = x.shape
    V, D = tok_embed.shape
    M = pos_embed.shape[0]
    G = seg_embed.shape[0]
    B = Bt * S

    table = _build_table(tok_embed, pos_embed, seg_embed, gamma, beta)
    x_flat = x.reshape(B).astype(jnp.int32)
    seg_flat = seg.reshape(B).astype(jnp.int32)
    out_flat = _make_sc_gather(B, D, M, G, n_chunk=64)(x_flat, seg_flat, table)
    return out_flat.reshape(Bt, S, D)
